# baseline (speedup 1.0000x reference)
import jax
import jax.numpy as jnp
from jax import lax
from jax.experimental import pallas as pl
from jax.experimental.pallas import tpu as pltpu
from jax.experimental.pallas import tpu_sc as plsc

_B = 16384
_D = 32
_NC = 2
_NS = 16
_NW = _NC * _NS
_BPW = _B // _NW
_CHUNK = 128
_NCHUNK = _BPW // _CHUNK
_L = 16


def _body(emb_hbm, lab_hbm, cen_hbm, out_hbm, idx_v, emb_v, cen_v, acc_v, sem):
    wid = lax.axis_index("s") * _NC + lax.axis_index("c")
    base = wid * _BPW

    pltpu.sync_copy(lab_hbm.at[wid], idx_v)
    copies = [
        pltpu.async_copy(
            cen_hbm.at[idx_v.at[j]],
            cen_v.at[pl.ds(j * _CHUNK, _CHUNK)],
            sem,
        )
        for j in range(_NCHUNK)
    ]
    pltpu.sync_copy(emb_hbm.at[pl.ds(base, _BPW)], emb_v)
    for c in copies:
        c.wait()

    def row(i, acc):
        d0 = emb_v[i, pl.ds(0, _L)] - cen_v[i, pl.ds(0, _L)]
        d1 = emb_v[i, pl.ds(_L, _L)] - cen_v[i, pl.ds(_L, _L)]
        return acc + d0 * d0 + d1 * d1

    acc = lax.fori_loop(0, _BPW, row, jnp.zeros((_L,), jnp.float32))
    acc_v[...] = acc
    pltpu.sync_copy(acc_v, out_hbm.at[wid])


@jax.jit
def kernel(embeddings, labels, centers):
    labels = labels.astype(jnp.int32).reshape(_NW, _NCHUNK, _CHUNK)
    partials = pl.kernel(
        _body,
        mesh=plsc.VectorSubcoreMesh(core_axis_name="c", subcore_axis_name="s"),
        compiler_params=pltpu.CompilerParams(use_tc_tiling_on_sc=False),
        out_type=jax.ShapeDtypeStruct((_NW, _L), jnp.float32),
        scratch_types=[
            pltpu.VMEM((_NCHUNK, _CHUNK), jnp.int32),
            pltpu.VMEM((_BPW, _D), jnp.float32),
            pltpu.VMEM((_BPW, _D), jnp.float32),
            pltpu.VMEM((_L,), jnp.float32),
            pltpu.SemaphoreType.DMA,
        ],
    )(embeddings, labels, centers)
    return jnp.sum(partials) * (1.0 / (_B * _D))
